# Initial kernel scaffold; baseline (speedup 1.0000x reference)
import jax
import jax.numpy as jnp
from jax.experimental import pallas as pl


def kernel(pred_mask, positive_points, negative_points):
    raise NotImplementedError("write your pallas kernel here")


# speedup vs baseline: 1.6602x; 1.6602x over previous
import functools

import jax
import jax.numpy as jnp
from jax import lax
from jax.experimental import pallas as pl
from jax.experimental.pallas import tpu as pltpu
from jax.experimental.pallas import tpu_sc as plsc

B, H, W = 16, 512, 512
NPTS = 256
NW = 32
P = 2 * B * NPTS
PPW = P // NW
LANES = 16


def _sc_gather(mask_flat, xs, ys):
    mesh = plsc.VectorSubcoreMesh(core_axis_name="c", subcore_axis_name="s")

    @functools.partial(
        pl.kernel,
        out_type=jax.ShapeDtypeStruct((P,), jnp.float32),
        mesh=mesh,
        scratch_types=[
            pltpu.VMEM((PPW,), jnp.int32),
            pltpu.VMEM((PPW,), jnp.int32),
            pltpu.VMEM((PPW,), jnp.int32),
            pltpu.VMEM((PPW,), jnp.float32),
            pltpu.SemaphoreType.DMA,
        ],
    )
    def k(mask_hbm, xs_hbm, ys_hbm, out_hbm, xv, yv, idxv, valv, sem):
        wid = lax.axis_index("s") * 2 + lax.axis_index("c")
        base = wid * PPW
        bhw = (wid % B) * (H * W)
        pltpu.sync_copy(xs_hbm.at[pl.ds(base, PPW)], xv)
        pltpu.sync_copy(ys_hbm.at[pl.ds(base, PPW)], yv)
        for j in range(PPW // LANES):
            sl = pl.ds(j * LANES, LANES)
            idxv[sl] = yv[sl] * W + xv[sl] + bhw
        copies = [
            pltpu.async_copy(
                mask_hbm.at[idxv.at[pl.ds(t * 128, 128)]],
                valv.at[pl.ds(t * 128, 128)],
                sem,
            )
            for t in range(PPW // 128)
        ]
        for c in copies:
            c.wait()
        pltpu.sync_copy(valv, out_hbm.at[pl.ds(base, PPW)])

    return k(mask_flat, xs, ys)


def _tc_reduce_body(v_ref, o_ref):
    v = v_ref[:]
    t = (lax.broadcasted_iota(jnp.int32, v.shape, 0) < 32).astype(jnp.float32)
    loss = jnp.maximum(v, 0.0) - v * t + jnp.log1p(jnp.exp(-jnp.abs(v)))
    o_ref[0, 0] = jnp.sum(loss) * (1.0 / P)


def _tc_reduce(vals):
    return pl.pallas_call(
        _tc_reduce_body,
        out_shape=jax.ShapeDtypeStruct((1, 1), jnp.float32),
        in_specs=[pl.BlockSpec(memory_space=pltpu.VMEM)],
        out_specs=pl.BlockSpec(memory_space=pltpu.SMEM),
    )(vals)


def kernel(pred_mask, positive_points, negative_points):
    mask_flat = pred_mask.reshape(-1)
    pp = positive_points.astype(jnp.int32)
    np_ = negative_points.astype(jnp.int32)
    xs = jnp.concatenate([pp[:, :, 0].reshape(-1), np_[:, :, 0].reshape(-1)])
    ys = jnp.concatenate([pp[:, :, 1].reshape(-1), np_[:, :, 1].reshape(-1)])
    vals = _sc_gather(mask_flat, xs, ys)
    return _tc_reduce(vals.reshape(P // 128, 128))[0, 0]


# speedup vs baseline: 2.2094x; 1.3308x over previous
import functools

import jax
import jax.numpy as jnp
from jax import lax
from jax.experimental import pallas as pl
from jax.experimental.pallas import tpu as pltpu
from jax.experimental.pallas import tpu_sc as plsc

B, H, W = 16, 512, 512
NPTS = 256
NW = 32
P = 2 * B * NPTS
PPW = P // NW
LANES = 16


def _sc_gather(mask2d, xs, ys):
    mesh = plsc.VectorSubcoreMesh(core_axis_name="c", subcore_axis_name="s")
    half = PPW // 2

    @functools.partial(
        pl.kernel,
        out_type=jax.ShapeDtypeStruct((P,), jnp.float32),
        mesh=mesh,
        scratch_types=[
            pltpu.VMEM((PPW,), jnp.int32),
            pltpu.VMEM((PPW,), jnp.int32),
            pltpu.VMEM((PPW,), jnp.int32),
            pltpu.VMEM((half, W), jnp.float32),
            pltpu.VMEM((PPW,), jnp.float32),
            pltpu.SemaphoreType.DMA,
        ],
        compiler_params=pltpu.CompilerParams(
            use_tc_tiling_on_sc=True, needs_layout_passes=False
        ),
    )
    def k(mask_hbm, xs_hbm, ys_hbm, out_hbm, xv, yv, idxv, rowsv, valv, sem):
        wid = lax.axis_index("s") * 2 + lax.axis_index("c")
        base = wid * PPW
        brow = (wid % B) * H
        pltpu.sync_copy(xs_hbm.at[pl.ds(base, PPW)], xv)
        pltpu.sync_copy(ys_hbm.at[pl.ds(base, PPW)], yv)
        for j in range(PPW // LANES):
            sl = pl.ds(j * LANES, LANES)
            idxv[sl] = yv[sl] + brow
        iota = lax.iota(jnp.int32, LANES)
        for h in range(PPW // half):
            pltpu.async_copy(
                mask_hbm.at[idxv.at[pl.ds(h * half, half)]],
                rowsv,
                sem,
            ).wait()
            for j in range(half // LANES):
                sl = pl.ds(h * half + j * LANES, LANES)
                valv[sl] = plsc.load_gather(
                    rowsv, [j * LANES + iota, xv[sl]]
                )
        pltpu.sync_copy(valv, out_hbm.at[pl.ds(base, PPW)])

    return k(mask2d, xs, ys)


def _tc_reduce_body(v_ref, o_ref):
    v = v_ref[:]
    t = (lax.broadcasted_iota(jnp.int32, v.shape, 0) < 32).astype(jnp.float32)
    loss = jnp.maximum(v, 0.0) - v * t + jnp.log1p(jnp.exp(-jnp.abs(v)))
    o_ref[0, 0] = jnp.sum(loss) * (1.0 / P)


def _tc_reduce(vals):
    return pl.pallas_call(
        _tc_reduce_body,
        out_shape=jax.ShapeDtypeStruct((1, 1), jnp.float32),
        in_specs=[pl.BlockSpec(memory_space=pltpu.VMEM)],
        out_specs=pl.BlockSpec(memory_space=pltpu.SMEM),
    )(vals)


def kernel(pred_mask, positive_points, negative_points):
    pp = positive_points.astype(jnp.int32)
    np_ = negative_points.astype(jnp.int32)
    xs = jnp.concatenate([pp[:, :, 0].reshape(-1), np_[:, :, 0].reshape(-1)])
    ys = jnp.concatenate([pp[:, :, 1].reshape(-1), np_[:, :, 1].reshape(-1)])
    vals = _sc_gather(pred_mask.reshape(B * H, W), xs, ys)
    return _tc_reduce(vals.reshape(P // 128, 128))[0, 0]
